# baseline (speedup 1.0000x reference)
import functools

import jax
import jax.numpy as jnp
from jax import lax
from jax.experimental import pallas as pl
from jax.experimental.pallas import tpu as pltpu
from jax.experimental.pallas import tpu_sc as plsc

N_MID = 1000000
DIM = 64
SEQ = 50
BATCH = 4096

NC = 2
NS = 16
NW = NC * NS

ROWS = BATCH * SEQ
SEQ_PER_W = BATCH // NW
ROWS_PER_W = SEQ_PER_W * SEQ
SEQ_PER_CHUNK = 16
CHUNK = SEQ_PER_CHUNK * SEQ
N_CHUNKS = ROWS_PER_W // CHUNK
G = 100
NG = CHUNK // G
LANES = 16
DSUB = DIM // LANES


def _sc_kernel(item_hbm, table_hbm, pos_hbm, out_hbm, idx_v, rows_v, pos_v, sem):
    wid = lax.axis_index("s") * NC + lax.axis_index("c")

    pltpu.sync_copy(pos_hbm, pos_v)

    def chunk_body(i, carry):
        cid = wid * N_CHUNKS + i
        row0 = cid * CHUNK

        pltpu.sync_copy(item_hbm.at[cid], idx_v)

        copies = []
        for j in range(NG):
            copies.append(
                pltpu.async_copy(
                    table_hbm.at[idx_v.at[j]],
                    rows_v.at[pl.ds(j * G, G)],
                    sem,
                )
            )
        for c in copies:
            c.wait()

        def s_body(s, c1):
            pv = [pos_v[s, pl.ds(LANES * d, LANES)] for d in range(DSUB)]

            def b_body(b, c2):
                r = b * SEQ + s
                for d in range(DSUB):
                    rows_v[r, pl.ds(LANES * d, LANES)] += pv[d]
                return c2

            return lax.fori_loop(0, SEQ_PER_CHUNK, b_body, c1)

        lax.fori_loop(0, SEQ, s_body, 0)

        pltpu.sync_copy(rows_v, out_hbm.at[pl.ds(row0, CHUNK)])
        return carry

    lax.fori_loop(0, N_CHUNKS, chunk_body, 0)


def kernel(item, nbr_mask, i_ids, item_input_lookup, position_embedding):
    del nbr_mask, i_ids
    item_chunks = item.reshape(NW * N_CHUNKS, NG, G)
    pos2d = position_embedding.reshape(SEQ, DIM)

    mesh = plsc.VectorSubcoreMesh(core_axis_name="c", subcore_axis_name="s")
    run = functools.partial(
        pl.kernel,
        mesh=mesh,
        out_type=jax.ShapeDtypeStruct((ROWS, DIM), jnp.float32),
        scratch_types=[
            pltpu.VMEM((NG, G), jnp.int32),
            pltpu.VMEM((CHUNK, DIM), jnp.float32),
            pltpu.VMEM((SEQ, DIM), jnp.float32),
            pltpu.SemaphoreType.DMA,
        ],
        compiler_params=pltpu.CompilerParams(use_tc_tiling_on_sc=False),
    )(_sc_kernel)
    out = run(item_chunks, item_input_lookup, pos2d)
    return out.reshape(BATCH, SEQ, DIM)


# speedup vs baseline: 1.0007x; 1.0007x over previous
import functools

import jax
import jax.numpy as jnp
from jax import lax
from jax.experimental import pallas as pl
from jax.experimental.pallas import tpu as pltpu
from jax.experimental.pallas import tpu_sc as plsc

N_MID = 1000000
DIM = 64
SEQ = 50
BATCH = 4096

NC = 2
NS = 16
NW = NC * NS

SEQ_PER_W = BATCH // NW
SEQ_PER_CHUNK = 16
N_CHUNKS = SEQ_PER_W // SEQ_PER_CHUNK
LANES = 16
DSUB = DIM // LANES


def _sc_kernel(item_hbm, table_hbm, pos_hbm, out_hbm, idx_v, rows_v, pos_v, sem):
    wid = lax.axis_index("s") * NC + lax.axis_index("c")

    pltpu.sync_copy(pos_hbm.at[0], pos_v)

    def chunk_body(i, carry):
        seq0 = wid * SEQ_PER_W + i * SEQ_PER_CHUNK

        pltpu.sync_copy(item_hbm.at[pl.ds(seq0, SEQ_PER_CHUNK)], idx_v)

        copies = []
        for j in range(SEQ_PER_CHUNK):
            copies.append(
                pltpu.async_copy(
                    table_hbm.at[idx_v.at[j]],
                    rows_v.at[j],
                    sem,
                )
            )
        for c in copies:
            c.wait()

        def s_body(s, c1):
            pv = [pos_v[s, pl.ds(LANES * d, LANES)] for d in range(DSUB)]

            def b_body(b, c2):
                for d in range(DSUB):
                    rows_v[b, s, pl.ds(LANES * d, LANES)] += pv[d]
                return c2

            return lax.fori_loop(0, SEQ_PER_CHUNK, b_body, c1)

        lax.fori_loop(0, SEQ, s_body, 0)

        pltpu.sync_copy(rows_v, out_hbm.at[pl.ds(seq0, SEQ_PER_CHUNK)])
        return carry

    lax.fori_loop(0, N_CHUNKS, chunk_body, 0)


def kernel(item, nbr_mask, i_ids, item_input_lookup, position_embedding):
    del nbr_mask, i_ids

    mesh = plsc.VectorSubcoreMesh(core_axis_name="c", subcore_axis_name="s")
    run = functools.partial(
        pl.kernel,
        mesh=mesh,
        out_type=jax.ShapeDtypeStruct((BATCH, SEQ, DIM), jnp.float32),
        scratch_types=[
            pltpu.VMEM((SEQ_PER_CHUNK, SEQ), jnp.int32),
            pltpu.VMEM((SEQ_PER_CHUNK, SEQ, DIM), jnp.float32),
            pltpu.VMEM((SEQ, DIM), jnp.float32),
            pltpu.SemaphoreType.DMA,
        ],
        compiler_params=pltpu.CompilerParams(use_tc_tiling_on_sc=False),
    )(_sc_kernel)
    return run(item, item_input_lookup, position_embedding)
